# baseline (speedup 1.0000x reference)
import functools
import math

import jax
import jax.numpy as jnp
from jax import lax
from jax.experimental import pallas as pl
from jax.experimental.pallas import tpu as pltpu
from jax.experimental.pallas import tpu_sc as plsc

D_MODEL = 64
SCALE = math.sqrt(D_MODEL)
NUM_WORKERS = 32
NBUF = 4
CHUNK = 80


def _emb_body(x_hbm, table_hbm, out_hbm, idx_all, r0, r1, r2, r3,
              gs0, gs1, gs2, gs3, ws0, ws1, ws2, ws3):
    rows = (r0, r1, r2, r3)
    gsem = (gs0, gs1, gs2, gs3)
    wsem = (ws0, ws1, ws2, ws3)
    wid = lax.axis_index("s") * 2 + lax.axis_index("c")
    per_w = x_hbm.shape[0] // NUM_WORKERS
    n = per_w // CHUNK
    base = wid * per_w

    pltpu.sync_copy(x_hbm.at[pl.ds(base, per_w)], idx_all)

    def issue_gather(g, s):
        def row16(t, c):
            iv = idx_all[pl.ds(g * CHUNK + t * 16, 16)]
            for l in range(16):
                pltpu.async_copy(
                    table_hbm.at[pl.ds(iv[l], 1)],
                    rows[s].at[pl.ds(t * 16 + l, 1)], gsem[s])
            return c

        lax.fori_loop(0, CHUNK // 16, row16, 0)

    def wait_gather(s):
        pltpu.make_async_copy(
            table_hbm.at[pl.ds(0, CHUNK)], rows[s], gsem[s]).wait()

    def issue_write(g, s):
        pltpu.async_copy(rows[s], out_hbm.at[pl.ds(base + g * CHUNK, CHUNK)], wsem[s])

    def wait_write(g, s):
        pltpu.make_async_copy(
            rows[s], out_hbm.at[pl.ds(base + g * CHUNK, CHUNK)], wsem[s]).wait()

    def scale(s):
        rbuf = rows[s]

        def body8(k, c):
            r = k * 8
            for i in range(8):
                for j in range(D_MODEL // 16):
                    sl = pl.ds(j * 16, 16)
                    rbuf[r + i, sl] = rbuf[r + i, sl] * SCALE
            return c

        lax.fori_loop(0, CHUNK // 8, body8, 0)

    for g in range(2):
        issue_gather(g, g)

    def quad(k, c):
        for b in range(NBUF):
            g = NBUF * k + b
            wait_gather(b)
            scale(b)
            issue_write(g, b)
            s_next = (b + 2) % NBUF

            @pl.when(g >= 2)
            def _():
                wait_write(g - 2, s_next)

            @pl.when(g + 2 < n)
            def _():
                issue_gather(g + 2, s_next)

        return c

    lax.fori_loop(0, n // NBUF, quad, 0)
    wait_write(n - 2, (n - 2) % NBUF)
    wait_write(n - 1, (n - 1) % NBUF)


@jax.jit
def _emb_call(table, idx):
    B = idx.shape[0]
    per_w = B // NUM_WORKERS
    mesh = plsc.VectorSubcoreMesh(core_axis_name="c", subcore_axis_name="s")
    fn = functools.partial(
        pl.kernel,
        mesh=mesh,
        out_type=jax.ShapeDtypeStruct((B, D_MODEL), jnp.float32),
        scratch_types=(
            [pltpu.VMEM((per_w,), jnp.int32)]
            + [pltpu.VMEM((CHUNK, D_MODEL), jnp.float32) for _ in range(NBUF)]
            + [pltpu.SemaphoreType.DMA for _ in range(2 * NBUF)]
        ),
    )(_emb_body)
    return fn(idx, table)


def kernel(lut_weight, x):
    xf = x.reshape(-1).astype(jnp.int32)
    out = _emb_call(lut_weight, xf)
    return out.reshape(x.shape + (D_MODEL,))


# speedup vs baseline: 1.0191x; 1.0191x over previous
import functools
import math

import jax
import jax.numpy as jnp
from jax import lax
from jax.experimental import pallas as pl
from jax.experimental.pallas import tpu as pltpu
from jax.experimental.pallas import tpu_sc as plsc

D_MODEL = 64
SCALE = math.sqrt(D_MODEL)
NUM_WORKERS = 32
NBUF = 5
CHUNK = 80


def _emb_body(x_hbm, table_hbm, out_hbm, idx_all, r0, r1, r2, r3, r4,
              gs0, gs1, gs2, gs3, gs4, ws0, ws1, ws2, ws3, ws4):
    rows = (r0, r1, r2, r3, r4)
    gsem = (gs0, gs1, gs2, gs3, gs4)
    wsem = (ws0, ws1, ws2, ws3, ws4)
    wid = lax.axis_index("s") * 2 + lax.axis_index("c")
    per_w = x_hbm.shape[0] // NUM_WORKERS
    n = per_w // CHUNK
    base = wid * per_w

    pltpu.sync_copy(x_hbm.at[pl.ds(base, per_w)], idx_all)

    def issue_gather(g, s):
        def row16(t, c):
            iv = idx_all[pl.ds(g * CHUNK + t * 16, 16)]
            for l in range(16):
                pltpu.async_copy(
                    table_hbm.at[pl.ds(iv[l], 1)],
                    rows[s].at[pl.ds(t * 16 + l, 1)], gsem[s])
            return c

        lax.fori_loop(0, CHUNK // 16, row16, 0)

    def wait_gather(s):
        pltpu.make_async_copy(
            table_hbm.at[pl.ds(0, CHUNK)], rows[s], gsem[s]).wait()

    def issue_write(g, s):
        pltpu.async_copy(rows[s], out_hbm.at[pl.ds(base + g * CHUNK, CHUNK)], wsem[s])

    def wait_write(g, s):
        pltpu.make_async_copy(
            rows[s], out_hbm.at[pl.ds(base + g * CHUNK, CHUNK)], wsem[s]).wait()

    def scale(s):
        rbuf = rows[s]

        def body8(k, c):
            r = k * 8
            for i in range(8):
                for j in range(D_MODEL // 16):
                    sl = pl.ds(j * 16, 16)
                    rbuf[r + i, sl] = rbuf[r + i, sl] * SCALE
            return c

        lax.fori_loop(0, CHUNK // 8, body8, 0)

    for g in range(NBUF - 1):
        issue_gather(g, g)

    def quad(k, c):
        for b in range(NBUF):
            g = NBUF * k + b
            wait_gather(b)
            scale(b)
            issue_write(g, b)
            s_next = (b + NBUF - 1) % NBUF

            @pl.when(g >= 1)
            def _():
                wait_write(g - 1, s_next)

            @pl.when(g + NBUF - 1 < n)
            def _():
                issue_gather(g + NBUF - 1, s_next)

        return c

    lax.fori_loop(0, n // NBUF, quad, 0)
    wait_write(n - 1, (n - 1) % NBUF)


@jax.jit
def _emb_call(table, idx):
    B = idx.shape[0]
    per_w = B // NUM_WORKERS
    mesh = plsc.VectorSubcoreMesh(core_axis_name="c", subcore_axis_name="s")
    fn = functools.partial(
        pl.kernel,
        mesh=mesh,
        out_type=jax.ShapeDtypeStruct((B, D_MODEL), jnp.float32),
        scratch_types=(
            [pltpu.VMEM((per_w,), jnp.int32)]
            + [pltpu.VMEM((CHUNK, D_MODEL), jnp.float32) for _ in range(NBUF)]
            + [pltpu.SemaphoreType.DMA for _ in range(2 * NBUF)]
        ),
    )(_emb_body)
    return fn(idx, table)


def kernel(lut_weight, x):
    xf = x.reshape(-1).astype(jnp.int32)
    out = _emb_call(lut_weight, xf)
    return out.reshape(x.shape + (D_MODEL,))


# speedup vs baseline: 1.0227x; 1.0034x over previous
import functools
import math

import jax
import jax.numpy as jnp
from jax import lax
from jax.experimental import pallas as pl
from jax.experimental.pallas import tpu as pltpu
from jax.experimental.pallas import tpu_sc as plsc

D_MODEL = 64
SCALE = math.sqrt(D_MODEL)
NUM_WORKERS = 32
NBUF = 4
CHUNK = 80


def _emb_body(x_hbm, table_hbm, out_hbm, idx_all, r0, r1, r2, r3,
              gs0, gs1, gs2, gs3, ws0, ws1, ws2, ws3):
    rows = (r0, r1, r2, r3)
    gsem = (gs0, gs1, gs2, gs3)
    wsem = (ws0, ws1, ws2, ws3)
    wid = lax.axis_index("s") * 2 + lax.axis_index("c")
    per_w = x_hbm.shape[0] // NUM_WORKERS
    n = per_w // CHUNK
    base = wid * per_w

    pltpu.sync_copy(x_hbm.at[pl.ds(base, per_w)], idx_all)

    def issue_gather(g, s):
        def row16(t, c):
            iv = idx_all[pl.ds(g * CHUNK + t * 16, 16)]
            for l in range(16):
                pltpu.async_copy(
                    table_hbm.at[pl.ds(iv[l], 1)],
                    rows[s].at[pl.ds(t * 16 + l, 1)], gsem[s])
            return c

        lax.fori_loop(0, CHUNK // 16, row16, 0)

    def wait_gather(s):
        pltpu.make_async_copy(
            table_hbm.at[pl.ds(0, CHUNK)], rows[s], gsem[s]).wait()

    def issue_write(g, s):
        pltpu.async_copy(rows[s], out_hbm.at[pl.ds(base + g * CHUNK, CHUNK)], wsem[s])

    def wait_write(g, s):
        pltpu.make_async_copy(
            rows[s], out_hbm.at[pl.ds(base + g * CHUNK, CHUNK)], wsem[s]).wait()

    def scale(s):
        rbuf = rows[s]

        def body8(k, c):
            r = k * 8
            for i in range(8):
                for j in range(D_MODEL // 16):
                    sl = pl.ds(j * 16, 16)
                    rbuf[r + i, sl] = rbuf[r + i, sl] * SCALE
            return c

        lax.fori_loop(0, CHUNK // 8, body8, 0)

    for g in range(3):
        issue_gather(g, g)

    def quad(k, c):
        for b in range(NBUF):
            g = NBUF * k + b
            wait_gather(b)
            scale(b)
            issue_write(g, b)
            s_next = (b + 3) % NBUF

            @pl.when(g >= 1)
            def _():
                wait_write(g - 1, s_next)

            @pl.when(g + 3 < n)
            def _():
                issue_gather(g + 3, s_next)

        return c

    lax.fori_loop(0, n // NBUF, quad, 0)
    wait_write(n - 1, (n - 1) % NBUF)


@jax.jit
def _emb_call(table, idx):
    B = idx.shape[0]
    per_w = B // NUM_WORKERS
    mesh = plsc.VectorSubcoreMesh(core_axis_name="c", subcore_axis_name="s")
    fn = functools.partial(
        pl.kernel,
        mesh=mesh,
        out_type=jax.ShapeDtypeStruct((B, D_MODEL), jnp.float32),
        scratch_types=(
            [pltpu.VMEM((per_w,), jnp.int32)]
            + [pltpu.VMEM((CHUNK, D_MODEL), jnp.float32) for _ in range(NBUF)]
            + [pltpu.SemaphoreType.DMA for _ in range(2 * NBUF)]
        ),
    )(_emb_body)
    return fn(idx, table)


def kernel(lut_weight, x):
    xf = x.reshape(-1).astype(jnp.int32)
    out = _emb_call(lut_weight, xf)
    return out.reshape(x.shape + (D_MODEL,))
